# baseline (speedup 1.0000x reference)
import functools
import math

import jax
import jax.numpy as jnp
from jax import lax
from jax.experimental import pallas as pl
from jax.experimental.pallas import tpu as pltpu
from jax.experimental.pallas import tpu_sc as plsc

D_MODEL = 128
SCALE = math.sqrt(D_MODEL)
NUM_CORES = 2
NUM_SUBCORES = 16
LANES = 16


@functools.lru_cache(maxsize=None)
def _make_emb(n_rows: int, chunk: int):
    n_workers = NUM_CORES * NUM_SUBCORES
    per_w = n_rows // n_workers
    n_chunks = per_w // chunk
    n_pairs = n_chunks // 2
    assert per_w * n_workers == n_rows and n_pairs * 2 * chunk == per_w

    mesh = plsc.VectorSubcoreMesh(
        core_axis_name="c", subcore_axis_name="s",
        num_cores=NUM_CORES, num_subcores=NUM_SUBCORES)

    @functools.partial(
        pl.kernel,
        out_type=jax.ShapeDtypeStruct((n_rows, D_MODEL), jnp.float32),
        mesh=mesh,
        scratch_types=[
            pltpu.VMEM((chunk,), jnp.int32),
            pltpu.VMEM((chunk,), jnp.int32),
            pltpu.VMEM((chunk, D_MODEL), jnp.float32),
            pltpu.VMEM((chunk, D_MODEL), jnp.float32),
            pltpu.VMEM((chunk, D_MODEL), jnp.float32),
            pltpu.VMEM((chunk, D_MODEL), jnp.float32),
            pltpu.SemaphoreType.DMA,
            pltpu.SemaphoreType.DMA,
            pltpu.SemaphoreType.DMA,
            pltpu.SemaphoreType.DMA,
            pltpu.SemaphoreType.DMA,
            pltpu.SemaphoreType.DMA,
        ],
    )
    def emb(x_hbm, table_hbm, out_hbm, idx_a, idx_b, rbuf_a, rbuf_b,
            obuf_a, obuf_b, isem0, isem1, gsem0, gsem1, osem0, osem1):
        idx2 = (idx_a, idx_b)
        rbuf = (rbuf_a, rbuf_b)
        obuf = (obuf_a, obuf_b)
        isem = (isem0, isem1)
        gsem = (gsem0, gsem1)
        osem = (osem0, osem1)
        wid = lax.axis_index("s") * NUM_CORES + lax.axis_index("c")
        base = wid * per_w

        def idx_copy(g, b):
            return pltpu.make_async_copy(
                x_hbm.at[pl.ds(base + g * chunk, chunk)], idx2[b], isem[b])

        def gather(b):
            return pltpu.make_async_copy(
                table_hbm.at[idx2[b]], rbuf[b], gsem[b])

        def out_copy(g, b):
            return pltpu.make_async_copy(
                obuf[b], out_hbm.at[pl.ds(base + g * chunk, chunk)],
                osem[b])

        pltpu.sync_copy(x_hbm.at[pl.ds(base, chunk)], idx2[0])
        gather(0).start()
        idx_copy(1, 1).start()

        def pair(p, carry):
            for b in (0, 1):
                g = 2 * p + b
                nb = 1 - b
                not_last = p < n_pairs - 1

                def launch_next():
                    idx_copy(g + 1, nb).wait()
                    gather(nb).start()
                if b == 0:
                    launch_next()
                else:
                    pl.when(not_last)(launch_next)

                gather(b).wait()
                pl.when(not_last)(lambda: idx_copy(g + 2, b).start())

                pl.when(p >= 1)(lambda: out_copy(g - 2, b).wait())

                def row_body(r, c):
                    for j in range(D_MODEL // LANES):
                        sl = pl.ds(j * LANES, LANES)
                        obuf[b][r, sl] = rbuf[b][r, sl] * SCALE
                    return c

                lax.fori_loop(0, chunk, row_body, 0, unroll=2)
                out_copy(g, b).start()
            return carry

        lax.fori_loop(0, n_pairs, pair, 0)
        out_copy(n_chunks - 2, 0).wait()
        out_copy(n_chunks - 1, 1).wait()

    return emb


def kernel(x, table):
    b0, b1 = x.shape
    n_rows = b0 * b1
    emb = _make_emb(n_rows, 200)
    out = emb(x.reshape(n_rows).astype(jnp.int32), table)
    return out.reshape(b0, b1, D_MODEL)
